# Initial kernel scaffold; baseline (speedup 1.0000x reference)
import jax
import jax.numpy as jnp
from jax.experimental import pallas as pl


def kernel(x, table):
    raise NotImplementedError("write your pallas kernel here")


# speedup vs baseline: 4.0819x; 4.0819x over previous
import functools

import jax
import jax.numpy as jnp
from jax import lax
from jax.experimental import pallas as pl
from jax.experimental.pallas import tpu as pltpu
from jax.experimental.pallas import tpu_sc as plsc


@functools.cache
def _make_gather(V, D, B):
    info = plsc.get_sparse_core_info()
    NC, NS = info.num_cores, info.num_subcores
    NW = NC * NS
    assert B % NW == 0
    b_per_w = B // NW
    C = 128
    assert b_per_w % C == 0
    n_chunks = b_per_w // C
    mesh = plsc.VectorSubcoreMesh(core_axis_name="c", subcore_axis_name="s")

    @functools.partial(
        pl.kernel,
        mesh=mesh,
        out_type=jax.ShapeDtypeStruct((B, D), jnp.float32),
        scratch_types=[
            pltpu.VMEM((b_per_w,), jnp.int32),
            pltpu.VMEM((C, D), jnp.float32),
            pltpu.SemaphoreType.DMA,
        ],
        compiler_params=pltpu.CompilerParams(use_tc_tiling_on_sc=False),
    )
    def k(table_hbm, idx_hbm, out_hbm, idx_v, rows_v, sem):
        wid = lax.axis_index("s") * NC + lax.axis_index("c")
        base = wid * b_per_w
        pltpu.sync_copy(idx_hbm.at[pl.ds(base, b_per_w)], idx_v)

        def body(i, carry):
            pltpu.async_copy(
                table_hbm.at[idx_v.at[pl.ds(i * C, C)]], rows_v, sem
            ).wait()
            pltpu.sync_copy(rows_v, out_hbm.at[pl.ds(base + i * C, C)])
            return carry

        lax.fori_loop(0, n_chunks, body, 0)

    return k


def kernel(x, table):
    B = x.shape[0] * x.shape[1]
    V, D = table.shape
    out = _make_gather(V, D, B)(table, x.reshape(B))
    return out.reshape(x.shape[0], x.shape[1], D)


# speedup vs baseline: 4.6991x; 1.1512x over previous
import functools

import jax
import jax.numpy as jnp
from jax import lax
from jax.experimental import pallas as pl
from jax.experimental.pallas import tpu as pltpu
from jax.experimental.pallas import tpu_sc as plsc


@functools.cache
def _make_gather(V, D, B):
    info = plsc.get_sparse_core_info()
    NC, NS = info.num_cores, info.num_subcores
    NW = NC * NS
    assert B % NW == 0
    b_per_w = B // NW
    C = 128
    assert b_per_w % C == 0
    n_chunks = b_per_w // C
    NBUF = 8
    G = 6
    assert G < NBUF <= n_chunks
    mesh = plsc.VectorSubcoreMesh(core_axis_name="c", subcore_axis_name="s")

    @functools.partial(
        pl.kernel,
        mesh=mesh,
        out_type=jax.ShapeDtypeStruct((B, D), jnp.float32),
        scratch_types=[
            pltpu.VMEM((b_per_w,), jnp.int32),
            pltpu.VMEM((NBUF, C, D), jnp.float32),
            pltpu.SemaphoreType.DMA((NBUF,)),
            pltpu.SemaphoreType.DMA((NBUF,)),
        ],
        compiler_params=pltpu.CompilerParams(use_tc_tiling_on_sc=False),
    )
    def k(table_hbm, idx_hbm, out_hbm, idx_v, rows_v, gsem, ssem):
        wid = lax.axis_index("s") * NC + lax.axis_index("c")
        base = wid * b_per_w
        pltpu.sync_copy(idx_hbm.at[pl.ds(base, b_per_w)], idx_v)

        def gather_start(j, b):
            pltpu.async_copy(
                table_hbm.at[idx_v.at[pl.ds(j * C, C)]], rows_v.at[b], gsem.at[b]
            )

        def gather_wait(j, b):
            pltpu.make_async_copy(
                table_hbm.at[idx_v.at[pl.ds(j * C, C)]], rows_v.at[b], gsem.at[b]
            ).wait()

        def store_start(i, b):
            pltpu.async_copy(
                rows_v.at[b], out_hbm.at[pl.ds(base + i * C, C)], ssem.at[b]
            )

        def store_wait(i, b):
            pltpu.make_async_copy(
                rows_v.at[b], out_hbm.at[pl.ds(base + i * C, C)], ssem.at[b]
            ).wait()

        for j in range(G):
            gather_start(j, j)

        def body(i, carry):
            b = lax.rem(i, NBUF)
            j = i + G
            bj = lax.rem(j, NBUF)

            @pl.when(j < n_chunks)
            def _():
                @pl.when(j >= NBUF)
                def _():
                    store_wait(j - NBUF, bj)
                gather_start(j, bj)

            gather_wait(i, b)
            store_start(i, b)
            return carry

        lax.fori_loop(0, n_chunks, body, 0)

        for u in range(NBUF):
            k_last = n_chunks - NBUF + ((u - (n_chunks - NBUF)) % NBUF)
            store_wait(k_last, u)

    return k


def kernel(x, table):
    B = x.shape[0] * x.shape[1]
    V, D = table.shape
    out = _make_gather(V, D, B)(table, x.reshape(B))
    return out.reshape(x.shape[0], x.shape[1], D)
